# Initial kernel scaffold; baseline (speedup 1.0000x reference)
import jax
import jax.numpy as jnp
from jax.experimental import pallas as pl


def kernel(x, edge_index, batch, W1_0, W1_r, b1, g1, be1, W2, b2, gn, bn, fc1_W, fc1_b, fc2_W, fc2_b):
    raise NotImplementedError("write your pallas kernel here")


# speedup vs baseline: 1.0963x; 1.0963x over previous
import functools

import jax
import jax.numpy as jnp
from jax import lax
from jax.experimental import pallas as pl
from jax.experimental.pallas import tpu as pltpu

N = 10000
E = 320000
D_IN = 128
H = 64
OUT = 16
G = 128
L = 5
N_PAD = 10016


def _bn(z, g, b):
    mu = jnp.mean(z, axis=0, keepdims=True)
    var = jnp.mean((z - mu) ** 2, axis=0, keepdims=True)
    return g * (z - mu) * lax.rsqrt(var + 1e-5) + b


def _dense0_body(x_ref, w_ref, out_ref):
    out_ref[:N, :] = jnp.dot(x_ref[:], w_ref[:],
                             preferred_element_type=jnp.float32)
    out_ref[N:, :] = jnp.zeros((N_PAD - N, H), jnp.float32)


def _layer_body(p_ref, agg_ref, b1_ref, g1_ref, be1_ref, w2_ref, b2_ref,
                gn_ref, bn_ref, w1n_ref, out_ref, *, last):
    z = p_ref[:N, :] + agg_ref[0, :N, :] + agg_ref[1, :N, :] + b1_ref[:]
    z = jnp.maximum(_bn(z, g1_ref[:], be1_ref[:]), 0.0)
    z = jnp.dot(z, w2_ref[:], preferred_element_type=jnp.float32) + b2_ref[:]
    if last:
        out_ref[:N, :] = z
    else:
        h = jnp.maximum(_bn(z, gn_ref[:], bn_ref[:]), 0.0)
        out_ref[:N, :] = jnp.dot(h, w1n_ref[:],
                                 preferred_element_type=jnp.float32)
    out_ref[N:, :] = jnp.zeros((N_PAD - N, H), jnp.float32)


def _pool_body(h_ref, batch_ref, fc1w_ref, fc1b_ref, fc2w_ref, fc2b_ref,
               out_ref):
    seg = batch_ref[:]
    onehot = (seg == lax.broadcasted_iota(jnp.int32, (N, G), 1))
    onehot = onehot.astype(jnp.float32)
    sums = lax.dot_general(onehot, h_ref[:N, :],
                           (((0,), (0,)), ((), ())),
                           preferred_element_type=jnp.float32)
    cnt = jnp.sum(onehot, axis=0, keepdims=True)
    pooled = sums / jnp.clip(cnt, 1.0, None).T
    y = jnp.maximum(
        jnp.dot(pooled, fc1w_ref[:], preferred_element_type=jnp.float32)
        + fc1b_ref[:], 0.0)
    out_ref[:] = jnp.dot(y, fc2w_ref[:],
                         preferred_element_type=jnp.float32) + fc2b_ref[:]


_dense0 = pl.pallas_call(
    _dense0_body,
    out_shape=jax.ShapeDtypeStruct((N_PAD, H), jnp.float32),
)

_pool = pl.pallas_call(
    _pool_body,
    out_shape=jax.ShapeDtypeStruct((G, OUT), jnp.float32),
)


def _agg_edges(p_pad, src, dst):
    a = jnp.zeros((N_PAD, H), jnp.float32).at[dst].add(
        p_pad[src], mode="drop", indices_are_sorted=False)
    return jnp.stack([a, jnp.zeros((N_PAD, H), jnp.float32)])


def kernel(x, edge_index, batch, W1_0, W1_r, b1, g1, be1, W2, b2, gn, bn,
           fc1_W, fc1_b, fc2_W, fc2_b):
    src = edge_index[0]
    dst = edge_index[1]

    p = _dense0(x, W1_0)
    for i in range(L):
        agg = _agg_edges(p, src, dst)
        last = i == L - 1
        w1n = W1_r[i] if not last else jnp.zeros((H, H), jnp.float32)
        layer = pl.pallas_call(
            functools.partial(_layer_body, last=last),
            out_shape=jax.ShapeDtypeStruct((N_PAD, H), jnp.float32),
        )
        p = layer(p, agg, b1[i].reshape(1, H), g1[i].reshape(1, H),
                  be1[i].reshape(1, H), W2[i], b2[i].reshape(1, H),
                  (gn[i] if not last else gn[0]).reshape(1, H),
                  (bn[i] if not last else bn[0]).reshape(1, H), w1n)

    return _pool(p, batch.reshape(N, 1), fc1_W, fc1_b.reshape(1, H),
                 fc2_W, fc2_b.reshape(1, OUT))


# speedup vs baseline: 4.9767x; 4.5398x over previous
import functools

import jax
import jax.numpy as jnp
from jax import lax
from jax.experimental import pallas as pl
from jax.experimental.pallas import tpu as pltpu
from jax.experimental.pallas import tpu_sc as plsc

N = 10000
E = 320000
D_IN = 128
H = 64
OUT = 16
G = 128
L = 5
N_PAD = 10112

NC, NS = 2, 16
CHUNK = 128
CHUNKS = 80
E_PAD = NC * NS * CHUNKS * CHUNK
ROWS_PT = N_PAD // NS


def _bn(z, g, b):
    mu = jnp.mean(z, axis=0, keepdims=True)
    var = jnp.mean((z - mu) ** 2, axis=0, keepdims=True)
    return g * (z - mu) * lax.rsqrt(var + 1e-5) + b


def _dense0_body(x_ref, w_ref, out_ref):
    out_ref[:N, :] = jnp.dot(x_ref[:], w_ref[:],
                             preferred_element_type=jnp.float32)
    out_ref[N:, :] = jnp.zeros((N_PAD - N, H), jnp.float32)


def _layer_body(p_ref, agg_ref, b1_ref, g1_ref, be1_ref, w2_ref, b2_ref,
                gn_ref, bn_ref, w1n_ref, out_ref, *, last):
    z = p_ref[:N, :] + agg_ref[0, :N, :] + agg_ref[1, :N, :] + b1_ref[:]
    z = jnp.maximum(_bn(z, g1_ref[:], be1_ref[:]), 0.0)
    z = jnp.dot(z, w2_ref[:], preferred_element_type=jnp.float32) + b2_ref[:]
    if last:
        out_ref[:N, :] = z
    else:
        h = jnp.maximum(_bn(z, gn_ref[:], bn_ref[:]), 0.0)
        out_ref[:N, :] = jnp.dot(h, w1n_ref[:],
                                 preferred_element_type=jnp.float32)
    out_ref[N:, :] = jnp.zeros((N_PAD - N, H), jnp.float32)


def _pool_body(h_ref, batch_ref, fc1w_ref, fc1b_ref, fc2w_ref, fc2b_ref,
               out_ref):
    seg = batch_ref[:]
    onehot = (seg == lax.broadcasted_iota(jnp.int32, (N, G), 1))
    onehot = onehot.astype(jnp.float32)
    sums = lax.dot_general(onehot, h_ref[:N, :],
                           (((0,), (0,)), ((), ())),
                           preferred_element_type=jnp.float32)
    cnt = jnp.sum(onehot, axis=0, keepdims=True)
    pooled = sums / jnp.clip(cnt, 1.0, None).T
    y = jnp.maximum(
        jnp.dot(pooled, fc1w_ref[:], preferred_element_type=jnp.float32)
        + fc1b_ref[:], 0.0)
    out_ref[:] = jnp.dot(y, fc2w_ref[:],
                         preferred_element_type=jnp.float32) + fc2b_ref[:]


_dense0 = pl.pallas_call(
    _dense0_body,
    out_shape=jax.ShapeDtypeStruct((N_PAD, H), jnp.float32),
)

_pool = pl.pallas_call(
    _pool_body,
    out_shape=jax.ShapeDtypeStruct((G, OUT), jnp.float32),
)


def _agg_body(src_hbm, dst_hbm, p_hbm, zeros_hbm, out_hbm,
              srcv, dstv, rows, gsem0, gsem1, acc):
    cid = lax.axis_index("c")
    sid = lax.axis_index("s")

    r0 = sid * ROWS_PT
    pltpu.sync_copy(zeros_hbm.at[pl.ds(r0, ROWS_PT)],
                    acc.at[pl.ds(r0, ROWS_PT)])
    pltpu.sync_copy(src_hbm.at[cid, sid], srcv)
    pltpu.sync_copy(dst_hbm.at[cid, sid], dstv)
    plsc.subcore_barrier()

    g0 = pltpu.async_copy(p_hbm.at[srcv.at[0]], rows.at[0], gsem0)
    g1 = pltpu.async_copy(p_hbm.at[srcv.at[1]], rows.at[1], gsem1)

    def step(k, _):
        t = 2 * k
        g0.wait()
        pltpu.sync_copy(rows.at[0], acc.at[dstv.at[t]], add=True)
        pltpu.async_copy(p_hbm.at[srcv.at[t + 2]], rows.at[0], gsem0)
        g1.wait()
        pltpu.sync_copy(rows.at[1], acc.at[dstv.at[t + 1]], add=True)
        pltpu.async_copy(p_hbm.at[srcv.at[t + 3]], rows.at[1], gsem1)
        return 0

    lax.fori_loop(0, CHUNKS // 2 - 1, step, 0)
    t = CHUNKS - 2
    g0.wait()
    pltpu.sync_copy(rows.at[0], acc.at[dstv.at[t]], add=True)
    g1.wait()
    pltpu.sync_copy(rows.at[1], acc.at[dstv.at[t + 1]], add=True)

    plsc.subcore_barrier()
    pltpu.sync_copy(acc.at[pl.ds(r0, ROWS_PT)],
                    out_hbm.at[cid, pl.ds(r0, ROWS_PT), :])


_agg_sc = functools.partial(
    pl.kernel,
    out_type=jax.ShapeDtypeStruct((NC, N_PAD, H), jnp.float32),
    mesh=plsc.VectorSubcoreMesh(core_axis_name="c", subcore_axis_name="s"),
    scratch_types=[
        pltpu.VMEM((CHUNKS, CHUNK), jnp.int32),
        pltpu.VMEM((CHUNKS, CHUNK), jnp.int32),
        pltpu.VMEM((2, CHUNK, H), jnp.float32),
        pltpu.SemaphoreType.DMA,
        pltpu.SemaphoreType.DMA,
        pltpu.VMEM_SHARED((N_PAD, H), jnp.float32),
    ],
    compiler_params=pltpu.CompilerParams(use_tc_tiling_on_sc=False),
)(_agg_body)


def _agg_edges(p_pad, src_t, dst_t, zeros_pad):
    return _agg_sc(src_t, dst_t, p_pad, zeros_pad)


def kernel(x, edge_index, batch, W1_0, W1_r, b1, g1, be1, W2, b2, gn, bn,
           fc1_W, fc1_b, fc2_W, fc2_b):
    pad = jnp.full((E_PAD - E,), N, jnp.int32)
    src_t = jnp.concatenate([edge_index[0], pad]).reshape(NC, NS, CHUNKS, CHUNK)
    dst_t = jnp.concatenate([edge_index[1], pad]).reshape(NC, NS, CHUNKS, CHUNK)
    zeros_pad = jnp.zeros((N_PAD, H), jnp.float32)

    p = _dense0(x, W1_0)
    for i in range(L):
        agg = _agg_edges(p, src_t, dst_t, zeros_pad)
        last = i == L - 1
        w1n = W1_r[i] if not last else jnp.zeros((H, H), jnp.float32)
        layer = pl.pallas_call(
            functools.partial(_layer_body, last=last),
            out_shape=jax.ShapeDtypeStruct((N_PAD, H), jnp.float32),
        )
        p = layer(p, agg, b1[i].reshape(1, H), g1[i].reshape(1, H),
                  be1[i].reshape(1, H), W2[i], b2[i].reshape(1, H),
                  (gn[i] if not last else gn[0]).reshape(1, H),
                  (bn[i] if not last else bn[0]).reshape(1, H), w1n)

    return _pool(p, batch.reshape(N, 1), fc1_W, fc1_b.reshape(1, H),
                 fc2_W, fc2_b.reshape(1, OUT))
